# Initial kernel scaffold; baseline (speedup 1.0000x reference)
import functools

import jax
import jax.numpy as jnp
from jax import lax
from jax.experimental import pallas as pl
from jax.experimental.pallas import tpu as pltpu
from jax.experimental.pallas import tpu_sc as plsc

N = 10000
E = 160000
IN_D = 256
HID_D = 256
OUT_D = 128

N_PAD = 10240
E_PAD = 163840
CHUNK = 128
ITERS = E_PAD // 16 // CHUNK
ROWS_PER_TILE = N_PAD // 16
DEG_W = 16
RB = 1024
GR = N_PAD // RB

_f32 = jnp.float32
_i32 = jnp.int32


def _mm2_body(x_ref, wl_ref, wr_ref, y_ref, z_ref):
    xb = x_ref[...]
    y_ref[...] = jnp.dot(xb, wl_ref[...], preferred_element_type=_f32)
    z_ref[...] = jnp.dot(xb, wr_ref[...], preferred_element_type=_f32)


def _mm2(xp, Wl, Wr, half):
    nh = Wl.shape[1] // half
    return pl.pallas_call(
        _mm2_body,
        grid=(GR, nh),
        in_specs=[
            pl.BlockSpec((RB, xp.shape[1]), lambda r, h: (r, 0)),
            pl.BlockSpec((Wl.shape[0], half), lambda r, h: (0, h)),
            pl.BlockSpec((Wr.shape[0], half), lambda r, h: (0, h)),
        ],
        out_specs=[
            pl.BlockSpec((RB, half), lambda r, h: (r + h * GR, 0)),
            pl.BlockSpec((RB, half), lambda r, h: (r, h)),
        ],
        out_shape=[
            jax.ShapeDtypeStruct((nh * N_PAD, half), _f32),
            jax.ShapeDtypeStruct((N_PAD, Wr.shape[1]), _f32),
        ],
    )(xp, Wl, Wr)


def _fuse1_body(agga_ref, aggb_ref, deg_ref, z1_ref, w2l_ref, w2r_ref, b1_ref,
                y2_ref, z2_ref):
    degc = jnp.clip(deg_ref[:, 0:1], 1.0, None)
    h = jnp.concatenate([agga_ref[...] / degc, aggb_ref[...] / degc], axis=1)
    h = jnp.maximum(h + z1_ref[...] + b1_ref[...], 0.0)
    y2_ref[...] = jnp.dot(h, w2l_ref[...], preferred_element_type=_f32)
    z2_ref[...] = jnp.dot(h, w2r_ref[...], preferred_element_type=_f32)


def _fuse1(agg_stk, deg, z1, W2l, W2r, b1):
    half = OUT_D // 2
    return pl.pallas_call(
        _fuse1_body,
        grid=(GR, 2),
        in_specs=[
            pl.BlockSpec((RB, HID_D // 2), lambda r, h: (r, 0)),
            pl.BlockSpec((RB, HID_D // 2), lambda r, h: (r + GR, 0)),
            pl.BlockSpec((RB, DEG_W), lambda r, h: (r, 0)),
            pl.BlockSpec((RB, HID_D), lambda r, h: (r, 0)),
            pl.BlockSpec((HID_D, half), lambda r, h: (0, h)),
            pl.BlockSpec((HID_D, half), lambda r, h: (0, h)),
            pl.BlockSpec((1, HID_D), lambda r, h: (0, 0)),
        ],
        out_specs=[
            pl.BlockSpec((RB, half), lambda r, h: (r + h * GR, 0)),
            pl.BlockSpec((RB, half), lambda r, h: (r, h)),
        ],
        out_shape=[
            jax.ShapeDtypeStruct((2 * N_PAD, half), _f32),
            jax.ShapeDtypeStruct((N_PAD, OUT_D), _f32),
        ],
    )(agg_stk, agg_stk, deg, z1, W2l, W2r, b1)


def _fuse2_body(a2a_ref, a2b_ref, deg_ref, z2_ref, b2_ref, out_ref):
    degc = jnp.clip(deg_ref[:, 0:1], 1.0, None)
    o = jnp.concatenate([a2a_ref[...] / degc, a2b_ref[...] / degc], axis=1)
    out_ref[...] = o + z2_ref[...] + b2_ref[...]


def _fuse2(agg2_stk, deg, z2, b2):
    half = OUT_D // 2
    return pl.pallas_call(
        _fuse2_body,
        grid=(GR,),
        in_specs=[
            pl.BlockSpec((RB, half), lambda r: (r, 0)),
            pl.BlockSpec((RB, half), lambda r: (r + GR, 0)),
            pl.BlockSpec((RB, DEG_W), lambda r: (r, 0)),
            pl.BlockSpec((RB, OUT_D), lambda r: (r, 0)),
            pl.BlockSpec((1, OUT_D), lambda r: (0, 0)),
        ],
        out_specs=pl.BlockSpec((RB, OUT_D), lambda r: (r, 0)),
        out_shape=jax.ShapeDtypeStruct((N_PAD, OUT_D), _f32),
    )(agg2_stk, agg2_stk, deg, z2, b2)


def _make_sc_seg_sum(D, with_deg):
    mesh = plsc.VectorSubcoreMesh(core_axis_name="c", subcore_axis_name="s")
    out_type = [jax.ShapeDtypeStruct((2 * N_PAD, D), _f32)]
    scratch = [
        pltpu.VMEM_SHARED((N_PAD, D), _f32),
        pltpu.VMEM((CHUNK,), _i32),
        pltpu.VMEM((CHUNK,), _i32),
        pltpu.VMEM((CHUNK, D), _f32),
        pltpu.VMEM((16, D), _f32),
        pltpu.SemaphoreType.DMA,
    ]
    if with_deg:
        out_type.append(jax.ShapeDtypeStruct((N_PAD, DEG_W), _f32))
        scratch += [
            pltpu.VMEM_SHARED((N_PAD, DEG_W), _f32),
            pltpu.VMEM((CHUNK, DEG_W), _f32),
            pltpu.VMEM((16, DEG_W), _f32),
        ]

    def body(table, src_hbm, dst_hbm, out_hbm, *rest):
        if with_deg:
            degout, acc, srcb, dstb, gbuf, zbuf, sem, dacc, onesb, zdbuf = rest
        else:
            acc, srcb, dstb, gbuf, zbuf, sem = rest
        cid = lax.axis_index("c")
        tid = lax.axis_index("s")
        rbase = tid * ROWS_PER_TILE

        zero16 = jnp.zeros((16,), _f32)
        for i in range(16):
            for j in range(D // 16):
                zbuf[i, pl.ds(j * 16, 16)] = zero16
        if with_deg:
            one16 = jnp.full((16,), 1.0, _f32)
            for i in range(CHUNK):
                onesb[i, pl.ds(0, 16)] = one16
            for i in range(16):
                zdbuf[i, pl.ds(0, 16)] = zero16

        def zero_acc(k, carry):
            pltpu.sync_copy(zbuf, acc.at[pl.ds(rbase + k * 16, 16)])
            return carry
        lax.fori_loop(0, ROWS_PER_TILE // 16, zero_acc, 0)
        if with_deg:
            @pl.when(cid == 0)
            def _():
                def zero_deg(k, carry):
                    pltpu.sync_copy(zdbuf, dacc.at[pl.ds(rbase + k * 16, 16)])
                    return carry
                lax.fori_loop(0, ROWS_PER_TILE // 16, zero_deg, 0)

        plsc.subcore_barrier()

        ebase = cid * E_PAD + tid * (E_PAD // 16)

        def edge_step(it, carry):
            off = ebase + it * CHUNK
            pltpu.sync_copy(src_hbm.at[pl.ds(off, CHUNK)], srcb)
            pltpu.sync_copy(dst_hbm.at[pl.ds(off, CHUNK)], dstb)
            pltpu.async_copy(table.at[srcb], gbuf, sem).wait()
            pltpu.sync_copy(gbuf, acc.at[dstb], add=True)
            if with_deg:
                @pl.when(cid == 0)
                def _():
                    pltpu.sync_copy(onesb, dacc.at[dstb], add=True)
            return carry
        lax.fori_loop(0, ITERS, edge_step, 0)

        plsc.subcore_barrier()

        for k in range(ROWS_PER_TILE // CHUNK):
            pltpu.sync_copy(acc.at[pl.ds(rbase + k * CHUNK, CHUNK)], gbuf)
            pltpu.sync_copy(
                gbuf, out_hbm.at[pl.ds(cid * N_PAD + rbase + k * CHUNK, CHUNK)])
        if with_deg:
            @pl.when(cid == 0)
            def _():
                for k in range(ROWS_PER_TILE // CHUNK):
                    pltpu.sync_copy(dacc.at[pl.ds(rbase + k * CHUNK, CHUNK)],
                                    onesb)
                    pltpu.sync_copy(onesb,
                                    degout.at[pl.ds(rbase + k * CHUNK, CHUNK)])

    return pl.kernel(body, out_type=out_type, mesh=mesh, scratch_types=scratch)


_sc_seg_sum_l1 = _make_sc_seg_sum(HID_D // 2, with_deg=True)
_sc_seg_sum_l2 = _make_sc_seg_sum(OUT_D // 2, with_deg=False)


def kernel(x, edge_index, W1l, b1, W1r, W2l, b2, W2r):
    pad_e = E_PAD - E
    src = edge_index[0].astype(_i32)
    dst = edge_index[1].astype(_i32)
    ar = jnp.arange(pad_e, dtype=_i32)
    src_p = jnp.concatenate([src, ar % N])
    dst_p = jnp.concatenate([dst, N + ar % (N_PAD - N)])
    src2 = jnp.concatenate([src_p, src_p + N_PAD])
    dst2 = jnp.concatenate([dst_p, dst_p])

    xp = jnp.pad(x, ((0, N_PAD - N), (0, 0)))
    b1r = b1.reshape(1, HID_D)
    b2r = b2.reshape(1, OUT_D)

    y1_stk, z1 = _mm2(xp, W1l, W1r, HID_D // 2)
    agg1_stk, deg = _sc_seg_sum_l1(y1_stk, src2, dst2)
    y2_stk, z2 = _fuse1(agg1_stk, deg, z1, W2l, W2r, b1r)
    agg2_stk = _sc_seg_sum_l2(y2_stk, src2, dst2)
    out = _fuse2(agg2_stk, deg, z2, b2r)
    return out[:N]


# speedup vs baseline: 4.9610x; 4.9610x over previous
import jax
import jax.numpy as jnp
from jax import lax
from jax.experimental import pallas as pl
from jax.experimental.pallas import tpu as pltpu
from jax.experimental.pallas import tpu_sc as plsc

N = 10000
E = 160000
IN_D = 256
HID_D = 256
OUT_D = 128

N_PAD = 10240
E_PAD = 163840
CHUNK = 128
ROWS_PER_TILE = N_PAD // 16
D = 128
RB = 1024
GR = N_PAD // RB

_f32 = jnp.float32
_i32 = jnp.int32


def _mm2_body(x_ref, wl_ref, wr_ref, y_ref, z_ref):
    xb = x_ref[...]
    y_ref[...] = jnp.dot(xb, wl_ref[...], preferred_element_type=_f32)
    z_ref[...] = jnp.dot(xb, wr_ref[...], preferred_element_type=_f32)


def _mm2(xp, Wl, Wr):
    nh = Wl.shape[1] // D
    return pl.pallas_call(
        _mm2_body,
        grid=(GR, nh),
        in_specs=[
            pl.BlockSpec((RB, xp.shape[1]), lambda r, h: (r, 0)),
            pl.BlockSpec((Wl.shape[0], D), lambda r, h: (0, h)),
            pl.BlockSpec((Wr.shape[0], D), lambda r, h: (0, h)),
        ],
        out_specs=[
            pl.BlockSpec((RB, D), lambda r, h: (r + h * GR, 0)),
            pl.BlockSpec((RB, D), lambda r, h: (r, h)),
        ],
        out_shape=[
            jax.ShapeDtypeStruct((nh * N_PAD, D), _f32),
            jax.ShapeDtypeStruct((N_PAD, Wr.shape[1]), _f32),
        ],
    )(xp, Wl, Wr)


def _fuse1_body(agga_ref, aggb_ref, dega_ref, degb_ref, z1_ref, w2l_ref,
                w2r_ref, b1_ref, y2_ref, z2_ref):
    degc = jnp.clip(dega_ref[:, 0:1] + degb_ref[:, 0:1], 1.0, None)
    h = jnp.concatenate([agga_ref[...] / degc, aggb_ref[...] / degc], axis=1)
    h = jnp.maximum(h + z1_ref[...] + b1_ref[...], 0.0)
    y2_ref[...] = jnp.dot(h, w2l_ref[...], preferred_element_type=_f32)
    z2_ref[...] = jnp.dot(h, w2r_ref[...], preferred_element_type=_f32)


def _fuse1(agg_stk, deg_stk, z1, W2l, W2r, b1):
    return pl.pallas_call(
        _fuse1_body,
        grid=(GR,),
        in_specs=[
            pl.BlockSpec((RB, D), lambda r: (r, 0)),
            pl.BlockSpec((RB, D), lambda r: (r + GR, 0)),
            pl.BlockSpec((RB, D), lambda r: (r, 0)),
            pl.BlockSpec((RB, D), lambda r: (r + GR, 0)),
            pl.BlockSpec((RB, HID_D), lambda r: (r, 0)),
            pl.BlockSpec((HID_D, OUT_D), lambda r: (0, 0)),
            pl.BlockSpec((HID_D, OUT_D), lambda r: (0, 0)),
            pl.BlockSpec((1, HID_D), lambda r: (0, 0)),
        ],
        out_specs=[
            pl.BlockSpec((RB, OUT_D), lambda r: (r, 0)),
            pl.BlockSpec((RB, OUT_D), lambda r: (r, 0)),
        ],
        out_shape=[
            jax.ShapeDtypeStruct((N_PAD, OUT_D), _f32),
            jax.ShapeDtypeStruct((N_PAD, OUT_D), _f32),
        ],
    )(agg_stk, agg_stk, deg_stk, deg_stk, z1, W2l, W2r, b1)


def _fuse2_body(a2a_ref, a2b_ref, dega_ref, degb_ref, z2_ref, b2_ref, out_ref):
    degc = jnp.clip(dega_ref[:, 0:1] + degb_ref[:, 0:1], 1.0, None)
    o = (a2a_ref[...] + a2b_ref[...]) / degc
    out_ref[...] = o + z2_ref[...] + b2_ref[...]


def _fuse2(agg2_stk, deg_stk, z2, b2):
    return pl.pallas_call(
        _fuse2_body,
        grid=(GR,),
        in_specs=[
            pl.BlockSpec((RB, OUT_D), lambda r: (r, 0)),
            pl.BlockSpec((RB, OUT_D), lambda r: (r + GR, 0)),
            pl.BlockSpec((RB, D), lambda r: (r, 0)),
            pl.BlockSpec((RB, D), lambda r: (r + GR, 0)),
            pl.BlockSpec((RB, OUT_D), lambda r: (r, 0)),
            pl.BlockSpec((1, OUT_D), lambda r: (0, 0)),
        ],
        out_specs=pl.BlockSpec((RB, OUT_D), lambda r: (r, 0)),
        out_shape=jax.ShapeDtypeStruct((N_PAD, OUT_D), _f32),
    )(agg2_stk, agg2_stk, deg_stk, deg_stk, z2, b2)


_MESH = plsc.VectorSubcoreMesh(core_axis_name="c", subcore_axis_name="s")


def _zero_fill(buf):
    zero16 = jnp.zeros((16,), _f32)
    for i in range(16):
        for j in range(D // 16):
            buf[i, pl.ds(j * 16, 16)] = zero16


def _zero_acc(acc, zbuf, rbase):
    def step(k, carry):
        pltpu.sync_copy(zbuf, acc.at[pl.ds(rbase + k * 16, 16)])
        return carry
    lax.fori_loop(0, ROWS_PER_TILE // 16, step, 0)


def _drain_acc(acc, gbuf, out_hbm, rbase, cid):
    for k in range(ROWS_PER_TILE // CHUNK):
        pltpu.sync_copy(acc.at[pl.ds(rbase + k * CHUNK, CHUNK)], gbuf)
        pltpu.sync_copy(
            gbuf, out_hbm.at[pl.ds(cid * N_PAD + rbase + k * CHUNK, CHUNK)])


def _make_sc_seg_sum(edges_per_core):
    scratch = [
        pltpu.VMEM_SHARED((N_PAD, D), _f32),
        pltpu.VMEM((CHUNK,), _i32),
        pltpu.VMEM((CHUNK,), _i32),
        pltpu.VMEM((CHUNK, D), _f32),
        pltpu.VMEM((16, D), _f32),
        pltpu.SemaphoreType.DMA,
    ]

    def body(table, src_hbm, dst_hbm, out_hbm, acc, srcb, dstb, gbuf, zbuf,
             sem):
        cid = lax.axis_index("c")
        tid = lax.axis_index("s")
        rbase = tid * ROWS_PER_TILE
        _zero_fill(zbuf)
        _zero_acc(acc, zbuf, rbase)
        plsc.subcore_barrier()

        epc_tile = edges_per_core // 16
        ebase = cid * edges_per_core + tid * epc_tile

        def edge_step(it, carry):
            off = ebase + it * CHUNK
            pltpu.sync_copy(src_hbm.at[pl.ds(off, CHUNK)], srcb)
            pltpu.sync_copy(dst_hbm.at[pl.ds(off, CHUNK)], dstb)
            pltpu.async_copy(table.at[srcb], gbuf, sem).wait()
            pltpu.sync_copy(gbuf, acc.at[dstb], add=True)
            return carry
        lax.fori_loop(0, epc_tile // CHUNK, edge_step, 0)

        plsc.subcore_barrier()
        _drain_acc(acc, gbuf, out_hbm, rbase, cid)

    return pl.kernel(body, out_type=jax.ShapeDtypeStruct((2 * N_PAD, D), _f32),
                     mesh=_MESH, scratch_types=scratch)


def _make_sc_deg(edges_per_core):
    scratch = [
        pltpu.VMEM_SHARED((N_PAD, D), _f32),
        pltpu.VMEM((CHUNK,), _i32),
        pltpu.VMEM((CHUNK, D), _f32),
        pltpu.VMEM((16, D), _f32),
    ]

    def body(dst_hbm, out_hbm, acc, dstb, onesb, zbuf):
        cid = lax.axis_index("c")
        tid = lax.axis_index("s")
        rbase = tid * ROWS_PER_TILE
        _zero_fill(zbuf)
        one16 = jnp.full((16,), 1.0, _f32)

        def fill_ones(i, carry):
            for j in range(D // 16):
                onesb[i, pl.ds(j * 16, 16)] = one16
            return carry
        lax.fori_loop(0, CHUNK, fill_ones, 0)
        _zero_acc(acc, zbuf, rbase)
        plsc.subcore_barrier()

        epc_tile = edges_per_core // 16
        ebase = cid * edges_per_core + tid * epc_tile

        def edge_step(it, carry):
            off = ebase + it * CHUNK
            pltpu.sync_copy(dst_hbm.at[pl.ds(off, CHUNK)], dstb)
            pltpu.sync_copy(onesb, acc.at[dstb], add=True)
            return carry
        lax.fori_loop(0, epc_tile // CHUNK, edge_step, 0)

        plsc.subcore_barrier()
        _drain_acc(acc, onesb, out_hbm, rbase, cid)

    return pl.kernel(body, out_type=jax.ShapeDtypeStruct((2 * N_PAD, D), _f32),
                     mesh=_MESH, scratch_types=scratch)


_sc_seg_sum_l1 = _make_sc_seg_sum(edges_per_core=E_PAD)
_sc_seg_sum_l2 = _make_sc_seg_sum(edges_per_core=E_PAD // 2)
_sc_deg = _make_sc_deg(edges_per_core=E_PAD // 2)


def kernel(x, edge_index, W1l, b1, W1r, W2l, b2, W2r):
    pad_e = E_PAD - E
    src = edge_index[0].astype(_i32)
    dst = edge_index[1].astype(_i32)
    ar = jnp.arange(pad_e, dtype=_i32)
    src_p = jnp.concatenate([src, ar % N])
    dst_p = jnp.concatenate([dst, N + ar % (N_PAD - N)])
    src2 = jnp.concatenate([src_p, src_p + N_PAD])
    dst2 = jnp.concatenate([dst_p, dst_p])

    xp = jnp.pad(x, ((0, N_PAD - N), (0, 0)))
    b1r = b1.reshape(1, HID_D)
    b2r = b2.reshape(1, OUT_D)

    deg_stk = _sc_deg(dst_p)
    y1_stk, z1 = _mm2(xp, W1l, W1r)
    agg1_stk = _sc_seg_sum_l1(y1_stk, src2, dst2)
    y2, z2 = _fuse1(agg1_stk, deg_stk, z1, W2l, W2r, b1r)
    agg2_stk = _sc_seg_sum_l2(y2, src_p, dst_p)
    out = _fuse2(agg2_stk, deg_stk, z2, b2r)
    return out[:N]


# speedup vs baseline: 6.7124x; 1.3530x over previous
import jax
import jax.numpy as jnp
from jax import lax
from jax.experimental import pallas as pl
from jax.experimental.pallas import tpu as pltpu
from jax.experimental.pallas import tpu_sc as plsc

N = 10000
E = 160000
IN_D = 256
HID_D = 256
OUT_D = 128

N_PAD = 10240
E_PAD = 163840
CHUNK = 128
ROWS_PER_TILE = N_PAD // 16
D = 128
RB = 1024
GR = N_PAD // RB

_f32 = jnp.float32
_i32 = jnp.int32


def _mm2_body(x_ref, wl_ref, wr_ref, y_ref, z_ref):
    xb = x_ref[...]
    y_ref[...] = jnp.dot(xb, wl_ref[...], preferred_element_type=_f32)
    z_ref[...] = jnp.dot(xb, wr_ref[...], preferred_element_type=_f32)


def _mm2(xp, Wl, Wr):
    nh = Wl.shape[1] // D
    return pl.pallas_call(
        _mm2_body,
        grid=(GR, nh),
        in_specs=[
            pl.BlockSpec((RB, xp.shape[1]), lambda r, h: (r, 0)),
            pl.BlockSpec((Wl.shape[0], D), lambda r, h: (0, h)),
            pl.BlockSpec((Wr.shape[0], D), lambda r, h: (0, h)),
        ],
        out_specs=[
            pl.BlockSpec((RB, D), lambda r, h: (r + h * GR, 0)),
            pl.BlockSpec((RB, D), lambda r, h: (r, h)),
        ],
        out_shape=[
            jax.ShapeDtypeStruct((nh * N_PAD, D), _f32),
            jax.ShapeDtypeStruct((N_PAD, Wr.shape[1]), _f32),
        ],
    )(xp, Wl, Wr)


def _fuse1_body(agga_ref, aggb_ref, dega_ref, degb_ref, z1_ref, w2l_ref,
                w2r_ref, b1_ref, y2_ref, z2_ref):
    degc = jnp.clip(dega_ref[:, 0:1] + degb_ref[:, 0:1], 1.0, None)
    h = jnp.concatenate([agga_ref[...] / degc, aggb_ref[...] / degc], axis=1)
    h = jnp.maximum(h + z1_ref[...] + b1_ref[...], 0.0)
    y2_ref[...] = jnp.dot(h, w2l_ref[...], preferred_element_type=_f32)
    z2_ref[...] = jnp.dot(h, w2r_ref[...], preferred_element_type=_f32)


def _fuse1(agg_stk, deg_stk, z1, W2l, W2r, b1):
    return pl.pallas_call(
        _fuse1_body,
        grid=(GR,),
        in_specs=[
            pl.BlockSpec((RB, D), lambda r: (r, 0)),
            pl.BlockSpec((RB, D), lambda r: (r + GR, 0)),
            pl.BlockSpec((RB, D), lambda r: (r, 0)),
            pl.BlockSpec((RB, D), lambda r: (r + GR, 0)),
            pl.BlockSpec((RB, HID_D), lambda r: (r, 0)),
            pl.BlockSpec((HID_D, OUT_D), lambda r: (0, 0)),
            pl.BlockSpec((HID_D, OUT_D), lambda r: (0, 0)),
            pl.BlockSpec((1, HID_D), lambda r: (0, 0)),
        ],
        out_specs=[
            pl.BlockSpec((RB, OUT_D), lambda r: (r, 0)),
            pl.BlockSpec((RB, OUT_D), lambda r: (r, 0)),
        ],
        out_shape=[
            jax.ShapeDtypeStruct((N_PAD, OUT_D), _f32),
            jax.ShapeDtypeStruct((N_PAD, OUT_D), _f32),
        ],
    )(agg_stk, agg_stk, deg_stk, deg_stk, z1, W2l, W2r, b1)


def _fuse2_body(a2a_ref, a2b_ref, dega_ref, degb_ref, z2_ref, b2_ref, out_ref):
    degc = jnp.clip(dega_ref[:, 0:1] + degb_ref[:, 0:1], 1.0, None)
    o = (a2a_ref[...] + a2b_ref[...]) / degc
    out_ref[...] = o + z2_ref[...] + b2_ref[...]


def _fuse2(agg2_stk, deg_stk, z2, b2):
    return pl.pallas_call(
        _fuse2_body,
        grid=(GR,),
        in_specs=[
            pl.BlockSpec((RB, OUT_D), lambda r: (r, 0)),
            pl.BlockSpec((RB, OUT_D), lambda r: (r + GR, 0)),
            pl.BlockSpec((RB, D), lambda r: (r, 0)),
            pl.BlockSpec((RB, D), lambda r: (r + GR, 0)),
            pl.BlockSpec((RB, OUT_D), lambda r: (r, 0)),
            pl.BlockSpec((1, OUT_D), lambda r: (0, 0)),
        ],
        out_specs=pl.BlockSpec((RB, OUT_D), lambda r: (r, 0)),
        out_shape=jax.ShapeDtypeStruct((N_PAD, OUT_D), _f32),
    )(agg2_stk, agg2_stk, deg_stk, deg_stk, z2, b2)


_MESH = plsc.VectorSubcoreMesh(core_axis_name="c", subcore_axis_name="s")


def _zero_fill(buf):
    zero16 = jnp.zeros((16,), _f32)
    for i in range(16):
        for j in range(D // 16):
            buf[i, pl.ds(j * 16, 16)] = zero16


def _zero_acc(acc, zbuf, rbase, zsem):
    descs = [pltpu.async_copy(zbuf, acc.at[pl.ds(rbase + k * 16, 16)], zsem)
             for k in range(ROWS_PER_TILE // 16)]
    for d in descs:
        d.wait()


def _drain_acc(acc, gbuf, out_hbm, rbase, cid):
    for k in range(ROWS_PER_TILE // CHUNK):
        pltpu.sync_copy(acc.at[pl.ds(rbase + k * CHUNK, CHUNK)], gbuf)
        pltpu.sync_copy(
            gbuf, out_hbm.at[pl.ds(cid * N_PAD + rbase + k * CHUNK, CHUNK)])


KBUF = 2
KBUF_DEG = 4


def _make_sc_seg_sum(edges_per_core):
    iters = edges_per_core // 16 // CHUNK
    scratch = [
        pltpu.VMEM_SHARED((N_PAD, D), _f32),
        [pltpu.VMEM((CHUNK,), _i32) for _ in range(KBUF)],
        [pltpu.VMEM((CHUNK,), _i32) for _ in range(KBUF)],
        [pltpu.VMEM((CHUNK, D), _f32) for _ in range(KBUF)],
        pltpu.VMEM((16, D), _f32),
        pltpu.SemaphoreType.DMA,
        [pltpu.SemaphoreType.DMA for _ in range(KBUF)],
        [pltpu.SemaphoreType.DMA for _ in range(KBUF)],
        [pltpu.SemaphoreType.DMA for _ in range(KBUF)],
        [pltpu.SemaphoreType.DMA for _ in range(KBUF)],
    ]

    def body(table, src_hbm, dst_hbm, out_hbm, acc, srcb, dstb, gbufs, zbuf,
             zsem, isems_s, isems_d, gsems, ssems):
        cid = lax.axis_index("c")
        tid = lax.axis_index("s")
        rbase = tid * ROWS_PER_TILE
        ebase = cid * edges_per_core + tid * (edges_per_core // 16)

        _zero_fill(zbuf)
        _zero_acc(acc, zbuf, rbase, zsem)
        plsc.subcore_barrier()

        def block(p, carry):
            offs = [ebase + (p * KBUF + b) * CHUNK for b in range(KBUF)]
            dis = [pltpu.async_copy(src_hbm.at[pl.ds(offs[b], CHUNK)],
                                    srcb[b], isems_s[b]) for b in range(KBUF)]
            did = [pltpu.async_copy(dst_hbm.at[pl.ds(offs[b], CHUNK)],
                                    dstb[b], isems_d[b]) for b in range(KBUF)]
            dg = []
            for b in range(KBUF):
                dis[b].wait()
                dg.append(pltpu.async_copy(table.at[srcb[b]], gbufs[b],
                                           gsems[b]))
            ds = []
            for b in range(KBUF):
                dg[b].wait()
                did[b].wait()
                ds.append(pltpu.async_copy(gbufs[b], acc.at[dstb[b]],
                                           ssems[b], add=True))
            for b in range(KBUF):
                ds[b].wait()
            return carry
        lax.fori_loop(0, iters // KBUF, block, 0)

        plsc.subcore_barrier()
        _drain_acc(acc, gbufs[0], out_hbm, rbase, cid)

    return pl.kernel(body, out_type=jax.ShapeDtypeStruct((2 * N_PAD, D), _f32),
                     mesh=_MESH, scratch_types=scratch)


def _make_sc_deg(edges_per_core):
    iters = edges_per_core // 16 // CHUNK
    scratch = [
        pltpu.VMEM_SHARED((N_PAD, D), _f32),
        [pltpu.VMEM((CHUNK,), _i32) for _ in range(KBUF_DEG)],
        pltpu.VMEM((CHUNK, D), _f32),
        pltpu.VMEM((16, D), _f32),
        pltpu.SemaphoreType.DMA,
        [pltpu.SemaphoreType.DMA for _ in range(KBUF_DEG)],
        [pltpu.SemaphoreType.DMA for _ in range(KBUF_DEG)],
    ]

    def body(dst_hbm, out_hbm, acc, dstb, onesb, zbuf, zsem, isems, ssems):
        cid = lax.axis_index("c")
        tid = lax.axis_index("s")
        rbase = tid * ROWS_PER_TILE
        ebase = cid * edges_per_core + tid * (edges_per_core // 16)

        _zero_fill(zbuf)
        one16 = jnp.full((16,), 1.0, _f32)

        def fill_ones(i, carry):
            for j in range(D // 16):
                onesb[i, pl.ds(j * 16, 16)] = one16
            return carry
        lax.fori_loop(0, CHUNK, fill_ones, 0)
        _zero_acc(acc, zbuf, rbase, zsem)
        plsc.subcore_barrier()

        def block(p, carry):
            offs = [ebase + (p * KBUF_DEG + b) * CHUNK for b in range(KBUF_DEG)]
            did = [pltpu.async_copy(dst_hbm.at[pl.ds(offs[b], CHUNK)],
                                    dstb[b], isems[b]) for b in range(KBUF_DEG)]
            ds = []
            for b in range(KBUF_DEG):
                did[b].wait()
                ds.append(pltpu.async_copy(onesb, acc.at[dstb[b]],
                                           ssems[b], add=True))
            for b in range(KBUF_DEG):
                ds[b].wait()
            return carry
        lax.fori_loop(0, iters // KBUF_DEG, block, 0)

        plsc.subcore_barrier()
        _drain_acc(acc, onesb, out_hbm, rbase, cid)

    return pl.kernel(body, out_type=jax.ShapeDtypeStruct((2 * N_PAD, D), _f32),
                     mesh=_MESH, scratch_types=scratch)


_sc_seg_sum_l1 = _make_sc_seg_sum(edges_per_core=E_PAD)
_sc_seg_sum_l2 = _make_sc_seg_sum(edges_per_core=E_PAD // 2)
_sc_deg = _make_sc_deg(edges_per_core=E_PAD // 2)


def kernel(x, edge_index, W1l, b1, W1r, W2l, b2, W2r):
    pad_e = E_PAD - E
    src = edge_index[0].astype(_i32)
    dst = edge_index[1].astype(_i32)
    ar = jnp.arange(pad_e, dtype=_i32)
    src_p = jnp.concatenate([src, ar % N])
    dst_p = jnp.concatenate([dst, N + ar % (N_PAD - N)])
    src2 = jnp.concatenate([src_p, src_p + N_PAD])
    dst2 = jnp.concatenate([dst_p, dst_p])

    xp = jnp.pad(x, ((0, N_PAD - N), (0, 0)))
    b1r = b1.reshape(1, HID_D)
    b2r = b2.reshape(1, OUT_D)

    deg_stk = _sc_deg(dst_p)
    y1_stk, z1 = _mm2(xp, W1l, W1r)
    agg1_stk = _sc_seg_sum_l1(y1_stk, src2, dst2)
    y2, z2 = _fuse1(agg1_stk, deg_stk, z1, W2l, W2r, b1r)
    agg2_stk = _sc_seg_sum_l2(y2, src_p, dst_p)
    out = _fuse2(agg2_stk, deg_stk, z2, b2r)
    return out[:N]


# speedup vs baseline: 6.8315x; 1.0177x over previous
import jax
import jax.numpy as jnp
from jax import lax
from jax.experimental import pallas as pl
from jax.experimental.pallas import tpu as pltpu
from jax.experimental.pallas import tpu_sc as plsc

N = 10000
E = 160000
IN_D = 256
HID_D = 256
OUT_D = 128

N_PAD = 10240
CHUNK = 120
KBUF = 3
E_PAD = 161280
ROWS_PER_TILE = N_PAD // 16
D = 128
RB = 1024
GR = N_PAD // RB

_f32 = jnp.float32
_i32 = jnp.int32


def _mm2_body(x_ref, wl_ref, wr_ref, y_ref, z_ref):
    xb = x_ref[...]
    y_ref[...] = jnp.dot(xb, wl_ref[...], preferred_element_type=_f32)
    z_ref[...] = jnp.dot(xb, wr_ref[...], preferred_element_type=_f32)


def _mm2(xp, Wl, Wr):
    nh = Wl.shape[1] // D
    return pl.pallas_call(
        _mm2_body,
        grid=(GR, nh),
        in_specs=[
            pl.BlockSpec((RB, xp.shape[1]), lambda r, h: (r, 0)),
            pl.BlockSpec((Wl.shape[0], D), lambda r, h: (0, h)),
            pl.BlockSpec((Wr.shape[0], D), lambda r, h: (0, h)),
        ],
        out_specs=[
            pl.BlockSpec((RB, D), lambda r, h: (r + h * GR, 0)),
            pl.BlockSpec((RB, D), lambda r, h: (r, h)),
        ],
        out_shape=[
            jax.ShapeDtypeStruct((nh * N_PAD, D), _f32),
            jax.ShapeDtypeStruct((N_PAD, Wr.shape[1]), _f32),
        ],
    )(xp, Wl, Wr)


def _fuse1_body(agga_ref, aggb_ref, dega_ref, degb_ref, z1_ref, w2l_ref,
                w2r_ref, b1_ref, y2_ref, z2_ref):
    degc = jnp.clip(dega_ref[:, 0:1] + degb_ref[:, 0:1], 1.0, None)
    h = jnp.concatenate([agga_ref[...] / degc, aggb_ref[...] / degc], axis=1)
    h = jnp.maximum(h + z1_ref[...] + b1_ref[...], 0.0)
    y2_ref[...] = jnp.dot(h, w2l_ref[...], preferred_element_type=_f32)
    z2_ref[...] = jnp.dot(h, w2r_ref[...], preferred_element_type=_f32)


def _fuse1(agg_stk, deg_stk, z1, W2l, W2r, b1):
    return pl.pallas_call(
        _fuse1_body,
        grid=(GR,),
        in_specs=[
            pl.BlockSpec((RB, D), lambda r: (r, 0)),
            pl.BlockSpec((RB, D), lambda r: (r + GR, 0)),
            pl.BlockSpec((RB, D), lambda r: (r, 0)),
            pl.BlockSpec((RB, D), lambda r: (r + GR, 0)),
            pl.BlockSpec((RB, HID_D), lambda r: (r, 0)),
            pl.BlockSpec((HID_D, OUT_D), lambda r: (0, 0)),
            pl.BlockSpec((HID_D, OUT_D), lambda r: (0, 0)),
            pl.BlockSpec((1, HID_D), lambda r: (0, 0)),
        ],
        out_specs=[
            pl.BlockSpec((RB, OUT_D), lambda r: (r, 0)),
            pl.BlockSpec((RB, OUT_D), lambda r: (r, 0)),
        ],
        out_shape=[
            jax.ShapeDtypeStruct((N_PAD, OUT_D), _f32),
            jax.ShapeDtypeStruct((N_PAD, OUT_D), _f32),
        ],
    )(agg_stk, agg_stk, deg_stk, deg_stk, z1, W2l, W2r, b1)


def _fuse2_body(a2a_ref, a2b_ref, dega_ref, degb_ref, z2_ref, b2_ref, out_ref):
    degc = jnp.clip(dega_ref[:, 0:1] + degb_ref[:, 0:1], 1.0, None)
    o = (a2a_ref[...] + a2b_ref[...]) / degc
    out_ref[...] = o + z2_ref[...] + b2_ref[...]


def _fuse2(agg2_stk, deg_stk, z2, b2):
    return pl.pallas_call(
        _fuse2_body,
        grid=(GR,),
        in_specs=[
            pl.BlockSpec((RB, OUT_D), lambda r: (r, 0)),
            pl.BlockSpec((RB, OUT_D), lambda r: (r + GR, 0)),
            pl.BlockSpec((RB, D), lambda r: (r, 0)),
            pl.BlockSpec((RB, D), lambda r: (r + GR, 0)),
            pl.BlockSpec((RB, OUT_D), lambda r: (r, 0)),
            pl.BlockSpec((1, OUT_D), lambda r: (0, 0)),
        ],
        out_specs=pl.BlockSpec((RB, OUT_D), lambda r: (r, 0)),
        out_shape=jax.ShapeDtypeStruct((N_PAD, OUT_D), _f32),
    )(agg2_stk, agg2_stk, deg_stk, deg_stk, z2, b2)


_MESH = plsc.VectorSubcoreMesh(core_axis_name="c", subcore_axis_name="s")


def _zero_acc(acc, zeros_hbm, rbase):
    pltpu.sync_copy(zeros_hbm.at[pl.ds(rbase, ROWS_PER_TILE)],
                    acc.at[pl.ds(rbase, ROWS_PER_TILE)])


def _drain_acc(acc, out_hbm, rbase, cid):
    pltpu.sync_copy(acc.at[pl.ds(rbase, ROWS_PER_TILE)],
                    out_hbm.at[pl.ds(cid * N_PAD + rbase, ROWS_PER_TILE)])


KBUF_DEG = 3


def _make_sc_seg_sum(edges_per_core):
    iters = edges_per_core // 16 // CHUNK
    scratch = [
        pltpu.VMEM_SHARED((N_PAD, D), _f32),
        [pltpu.VMEM((CHUNK,), _i32) for _ in range(KBUF)],
        [pltpu.VMEM((CHUNK,), _i32) for _ in range(KBUF)],
        [pltpu.VMEM((CHUNK, D), _f32) for _ in range(KBUF)],
        [pltpu.SemaphoreType.DMA for _ in range(KBUF)],
        [pltpu.SemaphoreType.DMA for _ in range(KBUF)],
        [pltpu.SemaphoreType.DMA for _ in range(KBUF)],
        [pltpu.SemaphoreType.DMA for _ in range(KBUF)],
    ]

    def body(table, src_hbm, dst_hbm, zeros_hbm, out_hbm, acc, srcb, dstb,
             gbufs, isems_s, isems_d, gsems, ssems):
        cid = lax.axis_index("c")
        tid = lax.axis_index("s")
        rbase = tid * ROWS_PER_TILE
        ebase = cid * edges_per_core + tid * (edges_per_core // 16)

        _zero_acc(acc, zeros_hbm, rbase)
        plsc.subcore_barrier()

        def block(p, carry):
            offs = [ebase + (p * KBUF + b) * CHUNK for b in range(KBUF)]
            dis = [pltpu.async_copy(src_hbm.at[pl.ds(offs[b], CHUNK)],
                                    srcb[b], isems_s[b]) for b in range(KBUF)]
            did = [pltpu.async_copy(dst_hbm.at[pl.ds(offs[b], CHUNK)],
                                    dstb[b], isems_d[b]) for b in range(KBUF)]
            dg = []
            for b in range(KBUF):
                dis[b].wait()
                dg.append(pltpu.async_copy(table.at[srcb[b]], gbufs[b],
                                           gsems[b]))
            ds = []
            for b in range(KBUF):
                dg[b].wait()
                did[b].wait()
                ds.append(pltpu.async_copy(gbufs[b], acc.at[dstb[b]],
                                           ssems[b], add=True))
            for b in range(KBUF):
                ds[b].wait()
            return carry
        lax.fori_loop(0, iters // KBUF, block, 0)

        plsc.subcore_barrier()
        _drain_acc(acc, out_hbm, rbase, cid)

    return pl.kernel(body, out_type=jax.ShapeDtypeStruct((2 * N_PAD, D), _f32),
                     mesh=_MESH, scratch_types=scratch)


def _make_sc_deg(edges_per_core):
    iters = edges_per_core // 16 // CHUNK
    scratch = [
        pltpu.VMEM_SHARED((N_PAD, D), _f32),
        [pltpu.VMEM((CHUNK,), _i32) for _ in range(KBUF_DEG)],
        pltpu.VMEM((CHUNK, D), _f32),
        [pltpu.SemaphoreType.DMA for _ in range(KBUF_DEG)],
        [pltpu.SemaphoreType.DMA for _ in range(KBUF_DEG)],
    ]

    def body(dst_hbm, zeros_hbm, out_hbm, acc, dstb, onesb, isems, ssems):
        cid = lax.axis_index("c")
        tid = lax.axis_index("s")
        rbase = tid * ROWS_PER_TILE
        ebase = cid * edges_per_core + tid * (edges_per_core // 16)

        one16 = jnp.full((16,), 1.0, _f32)

        def fill_ones(i, carry):
            for j in range(D // 16):
                onesb[i, pl.ds(j * 16, 16)] = one16
            return carry
        lax.fori_loop(0, CHUNK, fill_ones, 0)
        _zero_acc(acc, zeros_hbm, rbase)
        plsc.subcore_barrier()

        def block(p, carry):
            offs = [ebase + (p * KBUF_DEG + b) * CHUNK
                    for b in range(KBUF_DEG)]
            did = [pltpu.async_copy(dst_hbm.at[pl.ds(offs[b], CHUNK)],
                                    dstb[b], isems[b])
                   for b in range(KBUF_DEG)]
            ds = []
            for b in range(KBUF_DEG):
                did[b].wait()
                ds.append(pltpu.async_copy(onesb, acc.at[dstb[b]],
                                           ssems[b], add=True))
            for b in range(KBUF_DEG):
                ds[b].wait()
            return carry
        lax.fori_loop(0, iters // KBUF_DEG, block, 0)

        plsc.subcore_barrier()
        _drain_acc(acc, out_hbm, rbase, cid)

    return pl.kernel(body, out_type=jax.ShapeDtypeStruct((2 * N_PAD, D), _f32),
                     mesh=_MESH, scratch_types=scratch)


_sc_seg_sum_l1 = _make_sc_seg_sum(edges_per_core=E_PAD)
_sc_seg_sum_l2 = _make_sc_seg_sum(edges_per_core=E_PAD // 2)
_sc_deg = _make_sc_deg(edges_per_core=E_PAD // 2)


def kernel(x, edge_index, W1l, b1, W1r, W2l, b2, W2r):
    pad_e = E_PAD - E
    src = edge_index[0].astype(_i32)
    dst = edge_index[1].astype(_i32)
    ar = jnp.arange(pad_e, dtype=_i32)
    src_p = jnp.concatenate([src, ar % N])
    dst_p = jnp.concatenate([dst, N + ar % (N_PAD - N)])
    src2 = jnp.concatenate([src_p, src_p + N_PAD])
    dst2 = jnp.concatenate([dst_p, dst_p])

    xp = jnp.pad(x, ((0, N_PAD - N), (0, 0)))
    b1r = b1.reshape(1, HID_D)
    b2r = b2.reshape(1, OUT_D)

    zeros = jnp.zeros((N_PAD, D), _f32)
    deg_stk = _sc_deg(dst_p, zeros)
    y1_stk, z1 = _mm2(xp, W1l, W1r)
    agg1_stk = _sc_seg_sum_l1(y1_stk, src2, dst2, zeros)
    y2, z2 = _fuse1(agg1_stk, deg_stk, z1, W2l, W2r, b1r)
    agg2_stk = _sc_seg_sum_l2(y2, src_p, dst_p, zeros)
    out = _fuse2(agg2_stk, deg_stk, z2, b2r)
    return out[:N]


# speedup vs baseline: 7.0511x; 1.0321x over previous
import jax
import jax.numpy as jnp
from jax import lax
from jax.experimental import pallas as pl
from jax.experimental.pallas import tpu as pltpu
from jax.experimental.pallas import tpu_sc as plsc

N = 10000
E = 160000
IN_D = 256
HID_D = 256
OUT_D = 128

N_PAD = 10240
CHUNK = 120
KBUF = 3
E_PAD = 161280
ROWS_PER_TILE = N_PAD // 16
D = 128
RB = 1024
GR = N_PAD // RB

_f32 = jnp.float32
_i32 = jnp.int32


def _mm2_body(x_ref, wl_ref, wr_ref, y_ref, z_ref):
    xb = x_ref[...]
    y_ref[...] = jnp.dot(xb, wl_ref[...], preferred_element_type=_f32)
    z_ref[...] = jnp.dot(xb, wr_ref[...], preferred_element_type=_f32)


def _mm2(xp, Wl, Wr):
    nh = Wl.shape[1] // D
    return pl.pallas_call(
        _mm2_body,
        grid=(GR, nh),
        in_specs=[
            pl.BlockSpec((RB, xp.shape[1]), lambda r, h: (r, 0)),
            pl.BlockSpec((Wl.shape[0], D), lambda r, h: (0, h)),
            pl.BlockSpec((Wr.shape[0], D), lambda r, h: (0, h)),
        ],
        out_specs=[
            pl.BlockSpec((RB, D), lambda r, h: (r + h * GR, 0)),
            pl.BlockSpec((RB, D), lambda r, h: (r, h)),
        ],
        out_shape=[
            jax.ShapeDtypeStruct((nh * N_PAD, D), _f32),
            jax.ShapeDtypeStruct((N_PAD, Wr.shape[1]), _f32),
        ],
    )(xp, Wl, Wr)


def _fuse1_body(agga_ref, aggb_ref, dega_ref, degb_ref, z1_ref, w2l_ref,
                w2r_ref, b1_ref, y2_ref, z2_ref):
    degc = jnp.clip(dega_ref[:, 0:1] + degb_ref[:, 0:1], 1.0, None)
    h = jnp.concatenate([agga_ref[...] / degc, aggb_ref[...] / degc], axis=1)
    h = jnp.maximum(h + z1_ref[...] + b1_ref[...], 0.0)
    y2_ref[...] = jnp.dot(h, w2l_ref[...], preferred_element_type=_f32)
    z2_ref[...] = jnp.dot(h, w2r_ref[...], preferred_element_type=_f32)


def _fuse1(agg_stk, deg_stk, z1, W2l, W2r, b1):
    return pl.pallas_call(
        _fuse1_body,
        grid=(GR,),
        in_specs=[
            pl.BlockSpec((RB, D), lambda r: (r, 0)),
            pl.BlockSpec((RB, D), lambda r: (r + GR, 0)),
            pl.BlockSpec((RB, D), lambda r: (r, 0)),
            pl.BlockSpec((RB, D), lambda r: (r + GR, 0)),
            pl.BlockSpec((RB, HID_D), lambda r: (r, 0)),
            pl.BlockSpec((HID_D, OUT_D), lambda r: (0, 0)),
            pl.BlockSpec((HID_D, OUT_D), lambda r: (0, 0)),
            pl.BlockSpec((1, HID_D), lambda r: (0, 0)),
        ],
        out_specs=[
            pl.BlockSpec((RB, OUT_D), lambda r: (r, 0)),
            pl.BlockSpec((RB, OUT_D), lambda r: (r, 0)),
        ],
        out_shape=[
            jax.ShapeDtypeStruct((N_PAD, OUT_D), _f32),
            jax.ShapeDtypeStruct((N_PAD, OUT_D), _f32),
        ],
    )(agg_stk, agg_stk, deg_stk, deg_stk, z1, W2l, W2r, b1)


def _fuse2_body(a2a_ref, a2b_ref, dega_ref, degb_ref, z2_ref, b2_ref, out_ref):
    degc = jnp.clip(dega_ref[:, 0:1] + degb_ref[:, 0:1], 1.0, None)
    o = (a2a_ref[...] + a2b_ref[...]) / degc
    out_ref[...] = o + z2_ref[...] + b2_ref[...]


def _fuse2(agg2_stk, deg_stk, z2, b2):
    return pl.pallas_call(
        _fuse2_body,
        grid=(GR,),
        in_specs=[
            pl.BlockSpec((RB, OUT_D), lambda r: (r, 0)),
            pl.BlockSpec((RB, OUT_D), lambda r: (r + GR, 0)),
            pl.BlockSpec((RB, D), lambda r: (r, 0)),
            pl.BlockSpec((RB, D), lambda r: (r + GR, 0)),
            pl.BlockSpec((RB, OUT_D), lambda r: (r, 0)),
            pl.BlockSpec((1, OUT_D), lambda r: (0, 0)),
        ],
        out_specs=pl.BlockSpec((RB, OUT_D), lambda r: (r, 0)),
        out_shape=jax.ShapeDtypeStruct((N, OUT_D), _f32),
    )(agg2_stk, agg2_stk, deg_stk, deg_stk, z2, b2)


_MESH = plsc.VectorSubcoreMesh(core_axis_name="c", subcore_axis_name="s")


def _zero_acc(acc, zeros_hbm, rbase):
    pltpu.sync_copy(zeros_hbm.at[pl.ds(rbase, ROWS_PER_TILE)],
                    acc.at[pl.ds(rbase, ROWS_PER_TILE)])


def _drain_acc(acc, out_hbm, rbase, cid):
    pltpu.sync_copy(acc.at[pl.ds(rbase, ROWS_PER_TILE)],
                    out_hbm.at[pl.ds(cid * N_PAD + rbase, ROWS_PER_TILE)])


KBUF_DEG = 3


def _make_sc_seg_sum(edges_per_core):
    iters = edges_per_core // 16 // CHUNK
    scratch = [
        pltpu.VMEM_SHARED((N_PAD, D), _f32),
        [pltpu.VMEM((CHUNK,), _i32) for _ in range(KBUF)],
        [pltpu.VMEM((CHUNK,), _i32) for _ in range(KBUF)],
        [pltpu.VMEM((CHUNK, D), _f32) for _ in range(KBUF)],
        [pltpu.SemaphoreType.DMA for _ in range(KBUF)],
        [pltpu.SemaphoreType.DMA for _ in range(KBUF)],
        [pltpu.SemaphoreType.DMA for _ in range(KBUF)],
        [pltpu.SemaphoreType.DMA for _ in range(KBUF)],
    ]

    def body(table, src_hbm, dst_hbm, zeros_hbm, out_hbm, acc, srcb, dstb,
             gbufs, isems_s, isems_d, gsems, ssems):
        cid = lax.axis_index("c")
        tid = lax.axis_index("s")
        rbase = tid * ROWS_PER_TILE
        ebase = cid * edges_per_core + tid * (edges_per_core // 16)

        _zero_acc(acc, zeros_hbm, rbase)
        plsc.subcore_barrier()

        def block(p, carry):
            offs = [ebase + (p * KBUF + b) * CHUNK for b in range(KBUF)]
            dis = [pltpu.async_copy(src_hbm.at[pl.ds(offs[b], CHUNK)],
                                    srcb[b], isems_s[b]) for b in range(KBUF)]
            did = [pltpu.async_copy(dst_hbm.at[pl.ds(offs[b], CHUNK)],
                                    dstb[b], isems_d[b]) for b in range(KBUF)]
            dg = []
            for b in range(KBUF):
                dis[b].wait()
                dg.append(pltpu.async_copy(table.at[srcb[b]], gbufs[b],
                                           gsems[b]))
            ds = []
            for b in range(KBUF):
                dg[b].wait()
                did[b].wait()
                ds.append(pltpu.async_copy(gbufs[b], acc.at[dstb[b]],
                                           ssems[b], add=True))
            for b in range(KBUF):
                ds[b].wait()
            return carry
        lax.fori_loop(0, iters // KBUF, block, 0)

        plsc.subcore_barrier()
        _drain_acc(acc, out_hbm, rbase, cid)

    return pl.kernel(body, out_type=jax.ShapeDtypeStruct((2 * N_PAD, D), _f32),
                     mesh=_MESH, scratch_types=scratch)


def _make_sc_deg(edges_per_core):
    iters = edges_per_core // 16 // CHUNK
    scratch = [
        pltpu.VMEM_SHARED((N_PAD, D), _f32),
        [pltpu.VMEM((CHUNK,), _i32) for _ in range(KBUF_DEG)],
        pltpu.VMEM((CHUNK, D), _f32),
        [pltpu.SemaphoreType.DMA for _ in range(KBUF_DEG)],
        [pltpu.SemaphoreType.DMA for _ in range(KBUF_DEG)],
    ]

    def body(dst_hbm, zeros_hbm, out_hbm, acc, dstb, onesb, isems, ssems):
        cid = lax.axis_index("c")
        tid = lax.axis_index("s")
        rbase = tid * ROWS_PER_TILE
        ebase = cid * edges_per_core + tid * (edges_per_core // 16)

        one16 = jnp.full((16,), 1.0, _f32)

        def fill_ones(i, carry):
            for j in range(D // 16):
                onesb[i, pl.ds(j * 16, 16)] = one16
            return carry
        lax.fori_loop(0, CHUNK, fill_ones, 0)
        _zero_acc(acc, zeros_hbm, rbase)
        plsc.subcore_barrier()

        def block(p, carry):
            offs = [ebase + (p * KBUF_DEG + b) * CHUNK
                    for b in range(KBUF_DEG)]
            did = [pltpu.async_copy(dst_hbm.at[pl.ds(offs[b], CHUNK)],
                                    dstb[b], isems[b])
                   for b in range(KBUF_DEG)]
            ds = []
            for b in range(KBUF_DEG):
                did[b].wait()
                ds.append(pltpu.async_copy(onesb, acc.at[dstb[b]],
                                           ssems[b], add=True))
            for b in range(KBUF_DEG):
                ds[b].wait()
            return carry
        lax.fori_loop(0, iters // KBUF_DEG, block, 0)

        plsc.subcore_barrier()
        _drain_acc(acc, out_hbm, rbase, cid)

    return pl.kernel(body, out_type=jax.ShapeDtypeStruct((2 * N_PAD, D), _f32),
                     mesh=_MESH, scratch_types=scratch)


_sc_seg_sum_l1 = _make_sc_seg_sum(edges_per_core=E_PAD)
_sc_seg_sum_l2 = _make_sc_seg_sum(edges_per_core=E_PAD // 2)
_sc_deg = _make_sc_deg(edges_per_core=E_PAD // 2)


def kernel(x, edge_index, W1l, b1, W1r, W2l, b2, W2r):
    pad_e = E_PAD - E
    src = edge_index[0].astype(_i32)
    dst = edge_index[1].astype(_i32)
    ar = jnp.arange(pad_e, dtype=_i32)
    src_p = jnp.concatenate([src, ar % N])
    dst_p = jnp.concatenate([dst, N + ar % (N_PAD - N)])
    src2 = jnp.concatenate([src_p, src_p + N_PAD])
    dst2 = jnp.concatenate([dst_p, dst_p])

    b1r = b1.reshape(1, HID_D)
    b2r = b2.reshape(1, OUT_D)

    zeros = jnp.zeros((N_PAD, D), _f32)
    deg_stk = _sc_deg(dst_p, zeros)
    y1_stk, z1 = _mm2(x, W1l, W1r)
    agg1_stk = _sc_seg_sum_l1(y1_stk, src2, dst2, zeros)
    y2, z2 = _fuse1(agg1_stk, deg_stk, z1, W2l, W2r, b1r)
    agg2_stk = _sc_seg_sum_l2(y2, src_p, dst_p, zeros)
    return _fuse2(agg2_stk, deg_stk, z2, b2r)


# speedup vs baseline: 7.1924x; 1.0200x over previous
import jax
import jax.numpy as jnp
from jax import lax
from jax.experimental import pallas as pl
from jax.experimental.pallas import tpu as pltpu
from jax.experimental.pallas import tpu_sc as plsc

N = 10000
E = 160000
IN_D = 256
HID_D = 256
OUT_D = 128

N_PAD = 10240
CHUNK = 120
KBUF = 3
E_PAD = 161280
ROWS_PER_TILE = N_PAD // 16
D = 128
RB = 1024
GR = N_PAD // RB

_f32 = jnp.float32
_i32 = jnp.int32


def _mm2_body(x_ref, wl_ref, wr_ref, y_ref, z_ref):
    xb = x_ref[...]
    y_ref[...] = jnp.dot(xb, wl_ref[...], preferred_element_type=_f32)
    z_ref[...] = jnp.dot(xb, wr_ref[...], preferred_element_type=_f32)


def _mm2(xp, Wl, Wr):
    nh = Wl.shape[1] // D
    return pl.pallas_call(
        _mm2_body,
        grid=(GR, nh),
        in_specs=[
            pl.BlockSpec((RB, xp.shape[1]), lambda r, h: (r, 0)),
            pl.BlockSpec((Wl.shape[0], D), lambda r, h: (0, h)),
            pl.BlockSpec((Wr.shape[0], D), lambda r, h: (0, h)),
        ],
        out_specs=[
            pl.BlockSpec((RB, D), lambda r, h: (r + h * GR, 0)),
            pl.BlockSpec((RB, D), lambda r, h: (r, h)),
        ],
        out_shape=[
            jax.ShapeDtypeStruct((nh * N_PAD, D), _f32),
            jax.ShapeDtypeStruct((N_PAD, Wr.shape[1]), _f32),
        ],
    )(xp, Wl, Wr)


def _fuse1_body(agga_ref, aggb_ref, dega_ref, degb_ref, z1_ref, w2l_ref,
                w2r_ref, b1_ref, y2_ref, z2_ref):
    degc = jnp.clip(dega_ref[:, 0:1] + degb_ref[:, 0:1], 1.0, None)
    h = jnp.concatenate([agga_ref[...] / degc, aggb_ref[...] / degc], axis=1)
    h = jnp.maximum(h + z1_ref[...] + b1_ref[...], 0.0)
    y2_ref[...] = jnp.dot(h, w2l_ref[...], preferred_element_type=_f32)
    z2_ref[...] = jnp.dot(h, w2r_ref[...], preferred_element_type=_f32)


def _fuse1(agg_stk, deg_stk, z1, W2l, W2r, b1):
    return pl.pallas_call(
        _fuse1_body,
        grid=(GR,),
        in_specs=[
            pl.BlockSpec((RB, D), lambda r: (r, 0)),
            pl.BlockSpec((RB, D), lambda r: (r + GR, 0)),
            pl.BlockSpec((RB, D), lambda r: (r, 0)),
            pl.BlockSpec((RB, D), lambda r: (r + GR, 0)),
            pl.BlockSpec((RB, HID_D), lambda r: (r, 0)),
            pl.BlockSpec((HID_D, OUT_D), lambda r: (0, 0)),
            pl.BlockSpec((HID_D, OUT_D), lambda r: (0, 0)),
            pl.BlockSpec((1, HID_D), lambda r: (0, 0)),
        ],
        out_specs=[
            pl.BlockSpec((RB, OUT_D), lambda r: (r, 0)),
            pl.BlockSpec((RB, OUT_D), lambda r: (r, 0)),
        ],
        out_shape=[
            jax.ShapeDtypeStruct((N_PAD, OUT_D), _f32),
            jax.ShapeDtypeStruct((N_PAD, OUT_D), _f32),
        ],
    )(agg_stk, agg_stk, deg_stk, deg_stk, z1, W2l, W2r, b1)


def _fuse2_body(a2a_ref, a2b_ref, dega_ref, degb_ref, z2_ref, b2_ref, out_ref):
    degc = jnp.clip(dega_ref[:, 0:1] + degb_ref[:, 0:1], 1.0, None)
    o = (a2a_ref[...] + a2b_ref[...]) / degc
    out_ref[...] = o + z2_ref[...] + b2_ref[...]


def _fuse2(agg2_stk, deg_stk, z2, b2):
    return pl.pallas_call(
        _fuse2_body,
        grid=(GR,),
        in_specs=[
            pl.BlockSpec((RB, OUT_D), lambda r: (r, 0)),
            pl.BlockSpec((RB, OUT_D), lambda r: (r + GR, 0)),
            pl.BlockSpec((RB, D), lambda r: (r, 0)),
            pl.BlockSpec((RB, D), lambda r: (r + GR, 0)),
            pl.BlockSpec((RB, OUT_D), lambda r: (r, 0)),
            pl.BlockSpec((1, OUT_D), lambda r: (0, 0)),
        ],
        out_specs=pl.BlockSpec((RB, OUT_D), lambda r: (r, 0)),
        out_shape=jax.ShapeDtypeStruct((N, OUT_D), _f32),
    )(agg2_stk, agg2_stk, deg_stk, deg_stk, z2, b2)


_MESH = plsc.VectorSubcoreMesh(core_axis_name="c", subcore_axis_name="s")


def _zero_acc(acc, zeros_hbm, rbase):
    pltpu.sync_copy(zeros_hbm.at[pl.ds(rbase, ROWS_PER_TILE)],
                    acc.at[pl.ds(rbase, ROWS_PER_TILE)])


def _drain_acc(acc, out_hbm, rbase, cid):
    pltpu.sync_copy(acc.at[pl.ds(rbase, ROWS_PER_TILE)],
                    out_hbm.at[pl.ds(cid * N_PAD + rbase, ROWS_PER_TILE)])


KBUF_DEG = 3


def _make_sc_seg_sum(edges_per_core, with_deg=False):
    iters = edges_per_core // 16 // CHUNK
    deg_iters = (E_PAD // 32) // CHUNK
    agg_t = jax.ShapeDtypeStruct((2 * N_PAD, D), _f32)
    out_type = [agg_t, agg_t] if with_deg else agg_t
    scratch = [
        pltpu.VMEM_SHARED((N_PAD, D), _f32),
        [pltpu.VMEM((CHUNK,), _i32) for _ in range(KBUF)],
        [pltpu.VMEM((CHUNK,), _i32) for _ in range(KBUF)],
        [pltpu.VMEM((CHUNK, D), _f32) for _ in range(KBUF)],
        [pltpu.SemaphoreType.DMA for _ in range(KBUF)],
        [pltpu.SemaphoreType.DMA for _ in range(KBUF)],
        [pltpu.SemaphoreType.DMA for _ in range(KBUF)],
        [pltpu.SemaphoreType.DMA for _ in range(KBUF)],
    ]

    def body(table, src_hbm, dst_hbm, zeros_hbm, *rest):
        if with_deg:
            out_hbm, deg_hbm, acc, srcb, dstb, gbufs, isems_s, isems_d, \
                gsems, ssems = rest
        else:
            out_hbm, acc, srcb, dstb, gbufs, isems_s, isems_d, gsems, \
                ssems = rest
        cid = lax.axis_index("c")
        tid = lax.axis_index("s")
        rbase = tid * ROWS_PER_TILE
        ebase = cid * edges_per_core + tid * (edges_per_core // 16)

        _zero_acc(acc, zeros_hbm, rbase)
        plsc.subcore_barrier()

        def block(p, carry):
            offs = [ebase + (p * KBUF + b) * CHUNK for b in range(KBUF)]
            dis = [pltpu.async_copy(src_hbm.at[pl.ds(offs[b], CHUNK)],
                                    srcb[b], isems_s[b]) for b in range(KBUF)]
            did = [pltpu.async_copy(dst_hbm.at[pl.ds(offs[b], CHUNK)],
                                    dstb[b], isems_d[b]) for b in range(KBUF)]
            dg = []
            for b in range(KBUF):
                dis[b].wait()
                dg.append(pltpu.async_copy(table.at[srcb[b]], gbufs[b],
                                           gsems[b]))
            ds = []
            for b in range(KBUF):
                dg[b].wait()
                did[b].wait()
                ds.append(pltpu.async_copy(gbufs[b], acc.at[dstb[b]],
                                           ssems[b], add=True))
            for b in range(KBUF):
                ds[b].wait()
            return carry
        lax.fori_loop(0, iters // KBUF, block, 0)

        plsc.subcore_barrier()
        _drain_acc(acc, out_hbm, rbase, cid)

        if with_deg:
            _zero_acc(acc, zeros_hbm, rbase)
            onesb = gbufs[0]
            one16 = jnp.full((16,), 1.0, _f32)

            def fill_ones(i, carry):
                for j in range(D // 16):
                    onesb[i, pl.ds(j * 16, 16)] = one16
                return carry
            lax.fori_loop(0, CHUNK, fill_ones, 0)
            plsc.subcore_barrier()

            debase = cid * (E_PAD // 2) + tid * (E_PAD // 32)

            def dblock(p, carry):
                offs = [debase + (p * KBUF + b) * CHUNK for b in range(KBUF)]
                did = [pltpu.async_copy(dst_hbm.at[pl.ds(offs[b], CHUNK)],
                                        dstb[b], isems_d[b])
                       for b in range(KBUF)]
                ds = []
                for b in range(KBUF):
                    did[b].wait()
                    ds.append(pltpu.async_copy(onesb, acc.at[dstb[b]],
                                               ssems[b], add=True))
                for b in range(KBUF):
                    ds[b].wait()
                return carry
            lax.fori_loop(0, deg_iters // KBUF, dblock, 0)

            plsc.subcore_barrier()
            _drain_acc(acc, deg_hbm, rbase, cid)

    return pl.kernel(body, out_type=out_type, mesh=_MESH,
                     scratch_types=scratch)


_sc_seg_sum_l1 = _make_sc_seg_sum(edges_per_core=E_PAD, with_deg=True)
_sc_seg_sum_l2 = _make_sc_seg_sum(edges_per_core=E_PAD // 2)


def kernel(x, edge_index, W1l, b1, W1r, W2l, b2, W2r):
    pad_e = E_PAD - E
    src = edge_index[0].astype(_i32)
    dst = edge_index[1].astype(_i32)
    ar = jnp.arange(pad_e, dtype=_i32)
    src_p = jnp.concatenate([src, ar % N])
    dst_p = jnp.concatenate([dst, N + ar % (N_PAD - N)])
    src2 = jnp.concatenate([src_p, src_p + N_PAD])
    dst2 = jnp.concatenate([dst_p, dst_p])

    b1r = b1.reshape(1, HID_D)
    b2r = b2.reshape(1, OUT_D)

    zeros = jnp.zeros((N_PAD, D), _f32)
    y1_stk, z1 = _mm2(x, W1l, W1r)
    agg1_stk, deg_stk = _sc_seg_sum_l1(y1_stk, src2, dst2, zeros)
    y2, z2 = _fuse1(agg1_stk, deg_stk, z1, W2l, W2r, b1r)
    agg2_stk = _sc_seg_sum_l2(y2, src_p, dst_p, zeros)
    return _fuse2(agg2_stk, deg_stk, z2, b2r)
